# Initial kernel scaffold; baseline (speedup 1.0000x reference)
import jax
import jax.numpy as jnp
from jax.experimental import pallas as pl


def kernel(x, edge_index, batch, W_in, b_in, W_conv, b_conv, gn_w, gn_b, gn_a, W_out, b_out):
    raise NotImplementedError("write your pallas kernel here")


# speedup vs baseline: 5.4259x; 5.4259x over previous
import functools

import jax
import jax.numpy as jnp
from jax import lax
from jax.experimental import pallas as pl
from jax.experimental.pallas import tpu as pltpu
from jax.experimental.pallas import tpu_sc as plsc

N = 10000
NP = 10240
E = 320000
B = 16
IN_C = 128
HID = 256
HH = 128
OUT_C = 128
EPS = 1e-5
NEG = -3.0e38

TILE = 1024
GRID = NP // TILE

EK = 128
NPASS = 4
ECH = 40
ECHB = 42
EPT = E // 16
EPP = EPT // NPASS
DCH = 80
DPT = E // 32
DUMMY = N
RPT = NP // 16

_sc_mesh = plsc.VectorSubcoreMesh(core_axis_name="c", subcore_axis_name="s",
                                  num_cores=2, num_subcores=16)


@functools.partial(
    pl.kernel,
    mesh=_sc_mesh,
    out_type=jax.ShapeDtypeStruct((32, RPT, 16), jnp.float32),
    scratch_types=[
        pltpu.VMEM((DCH, EK), jnp.int32),
        pltpu.VMEM((EK, 16), jnp.float32),
        pltpu.VMEM_SHARED((NP, 16), jnp.float32),
    ],
)
def _deg_sc(dstw, ones_h, zeros_h, out, dst_v, ones_v, acc):
    c = lax.axis_index("c")
    s = lax.axis_index("s")
    w = c * 16 + s
    pltpu.sync_copy(dstw.at[w], dst_v)
    pltpu.sync_copy(ones_h, ones_v)
    pltpu.sync_copy(zeros_h, acc.at[pl.ds(s * RPT, RPT)])
    plsc.subcore_barrier()

    def body(ch, carry):
        pltpu.sync_copy(ones_v, acc.at[dst_v.at[ch]], add=True)
        return carry

    lax.fori_loop(0, DCH, body, 0)
    plsc.subcore_barrier()
    pltpu.sync_copy(acc.at[pl.ds(s * RPT, RPT)], out.at[w])


@functools.partial(
    pl.kernel,
    mesh=_sc_mesh,
    out_type=jax.ShapeDtypeStruct((32, RPT, HH), jnp.float32),
    scratch_types=[
        pltpu.VMEM((ECHB, EK), jnp.int32),
        pltpu.VMEM((ECHB, EK), jnp.int32),
        pltpu.VMEM((2, EK, HH), jnp.float32),
        pltpu.VMEM_SHARED((NP, HH), jnp.float32),
        pltpu.SemaphoreType.DMA,
        pltpu.SemaphoreType.DMA,
    ],
)
def _scatter_sc(mt2, srcw, dstw, zeros_h, out, src_v, dst_v, rows_v, acc,
                sem0, sem1):
    c = lax.axis_index("c")
    s = lax.axis_index("s")
    w = c * 16 + s
    pltpu.sync_copy(zeros_h, acc.at[pl.ds(s * RPT, RPT)])
    plsc.subcore_barrier()
    sems = (sem0, sem1)

    for p in range(NPASS):
        pltpu.sync_copy(srcw.at[w * NPASS + p], src_v)
        pltpu.sync_copy(dstw.at[s * NPASS + p], dst_v)
        for b in range(2):
            pltpu.async_copy(mt2.at[src_v.at[b]], rows_v.at[b], sems[b])

        def body(i, carry):
            for b in range(2):
                ch = i * 2 + b
                pltpu.make_async_copy(mt2.at[src_v.at[ch]], rows_v.at[b],
                                      sems[b]).wait()
                pltpu.sync_copy(rows_v.at[b], acc.at[dst_v.at[ch]], add=True)
                pltpu.async_copy(mt2.at[src_v.at[ch + 2]], rows_v.at[b],
                                 sems[b])
            return carry

        lax.fori_loop(0, ECH // 2, body, 0)
        for b in range(2):
            pltpu.make_async_copy(mt2.at[src_v.at[ECH + b]], rows_v.at[b],
                                  sems[b]).wait()
    plsc.subcore_barrier()
    pltpu.sync_copy(acc.at[pl.ds(s * RPT, RPT)], out.at[w])


def _onehot(bvec):
    ids = lax.broadcasted_iota(jnp.int32, (bvec.shape[0], B), 1)
    return (bvec[:, None] == ids).astype(jnp.float32)


def _dot(a, b):
    return lax.dot_general(a, b, (((1,), (0,)), ((), ())),
                           preferred_element_type=jnp.float32)


def _dot_t(a, b):
    return lax.dot_general(a, b, (((1,), (1,)), ((), ())),
                           preferred_element_type=jnp.float32)


def _dot_cn(a, b):
    return lax.dot_general(a, b, (((0,), (0,)), ((), ())),
                           preferred_element_type=jnp.float32)


def _pre_body(x_ref, win_ref, bin_ref, wc0_ref, da_ref, db_ref, bat_ref,
              h0_ref, dinv_ref, mt_ref, cnt_ref):
    i = pl.program_id(0)
    cnt = da_ref[:, :1] + db_ref[:, :1]
    dinv = lax.rsqrt(cnt + 1.0)
    h = jnp.maximum(_dot_t(x_ref[...], win_ref[...]) + bin_ref[...], 0.0)
    h0_ref[...] = h
    dinv_ref[...] = jnp.broadcast_to(dinv, (TILE, 16))
    mt = _dot_t(h, wc0_ref[...]) * dinv
    mt_ref[...] = jnp.stack([mt[:, :HH], mt[:, HH:]], axis=0)
    oh = _onehot(bat_ref[0, 0, :])
    csum = jnp.sum(oh, axis=0)

    @pl.when(i == 0)
    def _():
        cnt_ref[...] = jnp.zeros_like(cnt_ref)

    cnt_ref[...] += jnp.broadcast_to(csum[:, None], (B, HH))


def _agg_body(s2_ref, mt2_ref, dinv_ref, bc_ref, bat_ref, agg_ref, gsum_ref):
    i = pl.program_id(0)
    S = jnp.concatenate([s2_ref[0] + mt2_ref[0], s2_ref[1] + mt2_ref[1]],
                        axis=1)
    agg = dinv_ref[:, :1] * S + bc_ref[...]
    agg_ref[...] = agg
    oh = _onehot(bat_ref[0, 0, :])

    @pl.when(i == 0)
    def _():
        gsum_ref[...] = jnp.zeros_like(gsum_ref)

    gsum_ref[...] += _dot_cn(oh, agg)


def _sub_body(agg_ref, gsum_ref, cnt_ref, ga_ref, bat_ref, sub_ref, vsum_ref):
    i = pl.program_id(0)
    cnt = jnp.maximum(cnt_ref[:, :1], 1.0)
    mean = gsum_ref[...] / cnt
    oh = _onehot(bat_ref[0, 0, :])
    sub = agg_ref[...] - ga_ref[...] * _dot(oh, mean)
    sub_ref[...] = sub

    @pl.when(i == 0)
    def _():
        vsum_ref[...] = jnp.zeros_like(vsum_ref)

    vsum_ref[...] += _dot_cn(oh, sub * sub)


def _next_body(sub_ref, vsum_ref, cnt_ref, gw_ref, gb_ref, hres_ref, dinv_ref,
               wc_ref, bat_ref, h_ref, mt_ref):
    cnt = jnp.maximum(cnt_ref[:, :1], 1.0)
    rstd = lax.rsqrt(vsum_ref[...] / cnt + EPS)
    oh = _onehot(bat_ref[0, 0, :])
    hn = gw_ref[...] * sub_ref[...] * _dot(oh, rstd) + gb_ref[...]
    h = jnp.maximum(hn, 0.0) + hres_ref[...]
    h_ref[...] = h
    mt = _dot_t(h, wc_ref[...]) * dinv_ref[:, :1]
    mt_ref[...] = jnp.stack([mt[:, :HH], mt[:, HH:]], axis=0)


def _pool_body(sub_ref, vsum_ref, cnt_ref, gw_ref, gb_ref, hres_ref, bat_ref,
               zsum_ref, zmax_ref):
    i = pl.program_id(0)
    cnt = jnp.maximum(cnt_ref[:, :1], 1.0)
    rstd = lax.rsqrt(vsum_ref[...] / cnt + EPS)
    bvec = bat_ref[0, 0, :]
    oh = _onehot(bvec)
    hn = gw_ref[...] * sub_ref[...] * _dot(oh, rstd) + gb_ref[...]
    h = jnp.maximum(hn, 0.0) + hres_ref[...]

    @pl.when(i == 0)
    def _():
        zsum_ref[...] = jnp.zeros_like(zsum_ref)
        zmax_ref[...] = jnp.full_like(zmax_ref, NEG)

    zsum_ref[...] += _dot_cn(oh, h)
    rows = []
    for g in range(B):
        mask = oh[:, g:g + 1] > 0.5
        rows.append(jnp.max(jnp.where(mask, h, NEG), axis=0, keepdims=True))
    zmax_ref[...] = jnp.maximum(zmax_ref[...], jnp.concatenate(rows, axis=0))


def _out_body(zsum_ref, zmax_ref, cnt_ref, wout_ref, bout_ref, out_ref):
    cnt = jnp.maximum(cnt_ref[:, :1], 1.0)
    zmean = zsum_ref[...] / cnt
    zmx = zmax_ref[...]
    zmx = jnp.where(zmx <= -1e38, 0.0, zmx)
    z = jnp.concatenate([zmean, zmx], axis=1)
    out_ref[...] = _dot_t(z, wout_ref[...]) + bout_ref[...]


def _full(shape):
    nd = len(shape)
    return pl.BlockSpec(shape, lambda i: (0,) * nd)


_row = pl.BlockSpec((TILE, HID), lambda i: (i, 0))
_rowh = pl.BlockSpec((TILE, 16), lambda i: (i, 0))
_rowx = pl.BlockSpec((TILE, IN_C), lambda i: (i, 0))
_half2 = pl.BlockSpec((2, TILE, HH), lambda i: (0, i, 0))
_bat = pl.BlockSpec((1, 1, TILE), lambda i: (i, 0, 0))
_acc16 = pl.BlockSpec((B, HID), lambda i: (0, 0))
_cnt16 = pl.BlockSpec((B, HH), lambda i: (0, 0))

_f32 = jnp.float32

_pre = pl.pallas_call(
    _pre_body, grid=(GRID,),
    in_specs=[_rowx, _full((HID, IN_C)), _full((1, HID)), _full((HID, HID)),
              _rowh, _rowh, _bat],
    out_specs=[_row, _rowh, _half2, _cnt16],
    out_shape=[jax.ShapeDtypeStruct((NP, HID), _f32),
               jax.ShapeDtypeStruct((NP, 16), _f32),
               jax.ShapeDtypeStruct((2, NP, HH), _f32),
               jax.ShapeDtypeStruct((B, HH), _f32)],
)

_agg = pl.pallas_call(
    _agg_body, grid=(GRID,),
    in_specs=[_half2, _half2, _rowh, _full((1, HID)), _bat],
    out_specs=[_row, _acc16],
    out_shape=[jax.ShapeDtypeStruct((NP, HID), _f32),
               jax.ShapeDtypeStruct((B, HID), _f32)],
)

_sub = pl.pallas_call(
    _sub_body, grid=(GRID,),
    in_specs=[_row, _acc16, _cnt16, _full((1, HID)), _bat],
    out_specs=[_row, _acc16],
    out_shape=[jax.ShapeDtypeStruct((NP, HID), _f32),
               jax.ShapeDtypeStruct((B, HID), _f32)],
)

_next = pl.pallas_call(
    _next_body, grid=(GRID,),
    in_specs=[_row, _acc16, _cnt16, _full((1, HID)), _full((1, HID)), _row,
              _rowh, _full((HID, HID)), _bat],
    out_specs=[_row, _half2],
    out_shape=[jax.ShapeDtypeStruct((NP, HID), _f32),
               jax.ShapeDtypeStruct((2, NP, HH), _f32)],
)

_pool = pl.pallas_call(
    _pool_body, grid=(GRID,),
    in_specs=[_row, _acc16, _cnt16, _full((1, HID)), _full((1, HID)), _row,
              _bat],
    out_specs=[_acc16, _acc16],
    out_shape=[jax.ShapeDtypeStruct((B, HID), _f32),
               jax.ShapeDtypeStruct((B, HID), _f32)],
)

_outk = pl.pallas_call(
    _out_body, grid=(1,),
    in_specs=[_acc16, _acc16, _cnt16, _full((OUT_C, 2 * HID)),
              _full((1, OUT_C))],
    out_specs=_full((B, OUT_C)),
    out_shape=jax.ShapeDtypeStruct((B, OUT_C), _f32),
)


def kernel(x, edge_index, batch, W_in, b_in, W_conv, b_conv, gn_w, gn_b, gn_a,
           W_out, b_out):
    src = edge_index[0].astype(jnp.int32)
    dst = edge_index[1].astype(jnp.int32)
    batch = batch.astype(jnp.int32)

    dd = jnp.concatenate(
        [dst.reshape(32, DPT), jnp.full((32, DCH * EK - DPT), DUMMY,
                                        jnp.int32)], axis=1).reshape(32, DCH, EK)
    ones_h = jnp.ones((EK, 16), _f32)
    zeros16 = jnp.zeros((RPT, 16), _f32)
    deg2 = _deg_sc(dd, ones_h, zeros16).reshape(2, NP, 16)

    x_pad = jnp.concatenate([x, jnp.zeros((NP - N, IN_C), _f32)], axis=0)
    batp = jnp.concatenate([batch, jnp.full((NP - N,), B, jnp.int32)])
    bat3 = batp.reshape(GRID, 1, TILE)
    h0, dinv2d, mt0, counts2d = _pre(x_pad, W_in, b_in.reshape(1, HID),
                                     W_conv[0], deg2[0], deg2[1], bat3)

    padw = ECHB * EK - EPP
    s16 = jnp.concatenate(
        [src.reshape(16, NPASS, EPP),
         jnp.zeros((16, NPASS, padw), jnp.int32)], axis=2).reshape(
             16 * NPASS, ECHB, EK)
    d16 = jnp.concatenate(
        [dst.reshape(16, NPASS, EPP),
         jnp.full((16, NPASS, padw), DUMMY, jnp.int32)], axis=2).reshape(
             16 * NPASS, ECHB, EK)
    srcw = jnp.concatenate([s16, s16 + NP], axis=0)
    dstw = d16
    zeros128 = jnp.zeros((RPT, HH), _f32)

    def edge_pass(mt):
        return _scatter_sc(mt.reshape(2 * NP, HH), srcw, dstw,
                           zeros128).reshape(2, NP, HH)

    s0 = edge_pass(mt0)
    agg0, gsum0 = _agg(s0, mt0, dinv2d, b_conv[0].reshape(1, HID), bat3)
    sub0, vsum0 = _sub(agg0, gsum0, counts2d, gn_a[0].reshape(1, HID), bat3)
    h1, mt1 = _next(sub0, vsum0, counts2d, gn_w[0].reshape(1, HID),
                    gn_b[0].reshape(1, HID), h0, dinv2d, W_conv[1], bat3)

    s1 = edge_pass(mt1)
    agg1, gsum1 = _agg(s1, mt1, dinv2d, b_conv[1].reshape(1, HID), bat3)
    sub1, vsum1 = _sub(agg1, gsum1, counts2d, gn_a[1].reshape(1, HID), bat3)
    zsum, zmax = _pool(sub1, vsum1, counts2d, gn_w[1].reshape(1, HID),
                       gn_b[1].reshape(1, HID), h1, bat3)

    return _outk(zsum, zmax, counts2d, W_out, b_out.reshape(1, OUT_C))
